# baseline (speedup 1.0000x reference)
import functools

import jax
import jax.numpy as jnp
from jax import lax
from jax.experimental import pallas as pl
from jax.experimental.pallas import tpu as pltpu
from jax.experimental.pallas import tpu_sc as plsc

_NC = 2
_NS = 16
_NW = _NC * _NS
_LANES = 16


def _mesh():
    return plsc.VectorSubcoreMesh(
        core_axis_name="c", subcore_axis_name="s", num_cores=_NC, num_subcores=_NS
    )


def _winner_tables(flat_idx, symbol_idx, neg1, n_rows):
    e = flat_idx.shape[0]
    chunk = e // _NW

    @functools.partial(
        pl.kernel,
        out_type=jax.ShapeDtypeStruct((_NW, n_rows), jnp.int32),
        mesh=_mesh(),
        compiler_params=pltpu.CompilerParams(needs_layout_passes=False),
        scratch_types=[
            pltpu.VMEM((chunk,), jnp.int32),
            pltpu.VMEM((chunk,), jnp.int32),
            pltpu.VMEM((n_rows,), jnp.int32),
            pltpu.VMEM((2 * _LANES,), jnp.int32),
        ],
    )
    def k(fi_hbm, si_hbm, neg1_hbm, tables_hbm, fi_v, si_v, tbl_v, shf_v):
        wid = lax.axis_index("s") * _NC + lax.axis_index("c")
        base = wid * chunk
        pltpu.sync_copy(fi_hbm.at[pl.ds(base, chunk)], fi_v)
        pltpu.sync_copy(si_hbm.at[pl.ds(base, chunk)], si_v)
        pltpu.sync_copy(neg1_hbm, tbl_v)
        lanes = lax.iota(jnp.int32, _LANES)
        shf_v[pl.ds(_LANES, _LANES)] = jnp.zeros((_LANES,), jnp.int32)

        def body(g, carry):
            off = g * _LANES
            fi = fi_v[pl.ds(off, _LANES)]
            si = si_v[pl.ds(off, _LANES)]
            key = fi * chunk + (off + lanes)
            skey, sval = plsc.sort_key_val(key, si)
            row = skey // chunk
            shf_v[pl.ds(0, _LANES)] = row
            nxt = shf_v[pl.ds(1, _LANES)]
            surv = (row != nxt) | (lanes == _LANES - 1)
            plsc.store_scatter(tbl_v, [row], sval, mask=surv)
            return carry

        lax.fori_loop(0, chunk // _LANES, body, 0)
        pltpu.sync_copy(tbl_v, tables_hbm.at[wid])

    return k(flat_idx, symbol_idx, neg1)


def _combine_tables(tables, n_rows):
    rows_per_w = n_rows // _NW

    @functools.partial(
        pl.kernel,
        out_type=(
            jax.ShapeDtypeStruct((n_rows,), jnp.int32),
            jax.ShapeDtypeStruct((n_rows,), jnp.int32),
        ),
        mesh=_mesh(),
        scratch_types=[
            pltpu.VMEM((_NW, rows_per_w), jnp.int32),
            pltpu.VMEM((rows_per_w,), jnp.int32),
            pltpu.VMEM((rows_per_w,), jnp.int32),
        ],
    )
    def k(tables_hbm, win_hbm, idx_hbm, tbuf, win_v, idx_v):
        wid = lax.axis_index("s") * _NC + lax.axis_index("c")
        rbase = wid * rows_per_w
        pltpu.sync_copy(tables_hbm.at[:, pl.ds(rbase, rows_per_w)], tbuf)

        def body(g, carry):
            acc = jnp.full((_LANES,), -1, jnp.int32)
            for c in range(_NW):
                t = tbuf[c, pl.ds(g * _LANES, _LANES)]
                acc = jnp.where(t >= 0, t, acc)
            win_v[pl.ds(g * _LANES, _LANES)] = acc
            idx_v[pl.ds(g * _LANES, _LANES)] = jnp.maximum(acc, 0)
            return carry

        lax.fori_loop(0, rows_per_w // _LANES, body, 0)
        pltpu.sync_copy(win_v, win_hbm.at[pl.ds(rbase, rows_per_w)])
        pltpu.sync_copy(idx_v, idx_hbm.at[pl.ds(rbase, rows_per_w)])

    return k(tables)


def _gather_symbols(symi, idx, n_rows, s_rows, w):
    rows_per_w = n_rows // _NW
    gb = 32
    ring = 8
    nb = rows_per_w // gb

    @functools.partial(
        pl.kernel,
        out_type=jax.ShapeDtypeStruct((n_rows, w), jnp.int32),
        mesh=_mesh(),
        scratch_types=[
            pltpu.VMEM((rows_per_w,), jnp.int32),
            [pltpu.VMEM((gb, w), jnp.int32) for _ in range(ring)],
            [pltpu.SemaphoreType.DMA for _ in range(ring)],
            pltpu.SemaphoreType.DMA,
        ],
    )
    def k(symi_hbm, idx_hbm, u_hbm, idx_v, bufs, gsems, osem):
        wid = lax.axis_index("s") * _NC + lax.axis_index("c")
        rbase = wid * rows_per_w
        pltpu.sync_copy(idx_hbm.at[pl.ds(rbase, rows_per_w)], idx_v)

        def gstart(g, j):
            pltpu.async_copy(
                symi_hbm.at[idx_v.at[pl.ds(g * gb, gb)]], bufs[j], gsems[j]
            )

        def gwait(j):
            pltpu.make_async_copy(
                symi_hbm.at[idx_v.at[pl.ds(0, gb)]], bufs[j], gsems[j]
            ).wait()

        def ostart(g, j):
            pltpu.async_copy(bufs[j], u_hbm.at[pl.ds(rbase + g * gb, gb)], osem)

        def owait(j):
            pltpu.make_async_copy(bufs[j], u_hbm.at[pl.ds(rbase, gb)], osem).wait()

        for j in range(ring - 1):
            gstart(j, j)

        def pbody(i, carry):
            for j in range(ring):
                g = ring * i + j
                jp = (j + ring - 1) % ring
                if j == 0:
                    @pl.when(i > 0)
                    def _():
                        owait(jp)
                else:
                    owait(jp)

                @pl.when(g + ring - 1 < nb)
                def _():
                    gstart(g + ring - 1, jp)

                gwait(j)
                ostart(g, j)
            return carry

        lax.fori_loop(0, nb // ring, pbody, 0)
        owait(ring - 1)

    return k(symi, idx)


def _fused_body(flat_ref, u_ref, win_ref, wtop_ref, wbot_ref, bias_ref, out_ref):
    prev = flat_ref[...]
    zh = (
        jnp.dot(prev.astype(jnp.bfloat16), wtop_ref[...], preferred_element_type=jnp.float32)
        + jnp.dot(u_ref[...], wbot_ref[...], preferred_element_type=jnp.float32)
        + bias_ref[...]
    )
    d = prev.shape[-1]
    z = jax.nn.sigmoid(zh[:, :d])
    h = jnp.tanh(zh[:, d:])
    blk = prev.shape[0]
    maskf = (win_ref[0, 0, :] >= 0).astype(jnp.float32).reshape(blk, 1)
    out_ref[...] = prev + (maskf * z) * (h - prev)


def _fused_update(flat, u, win_sym, w_top, w_bot, bias, blk=512):
    n, d = flat.shape
    grid = n // blk
    win3 = win_sym.reshape(grid, 1, blk)
    return pl.pallas_call(
        _fused_body,
        grid=(grid,),
        in_specs=[
            pl.BlockSpec((blk, d), lambda i: (i, 0)),
            pl.BlockSpec((blk, d), lambda i: (i, 0)),
            pl.BlockSpec((1, 1, blk), lambda i: (i, 0, 0)),
            pl.BlockSpec((d, 2 * d), lambda i: (0, 0)),
            pl.BlockSpec((d, 2 * d), lambda i: (0, 0)),
            pl.BlockSpec((1, 2 * d), lambda i: (0, 0)),
        ],
        out_specs=pl.BlockSpec((blk, d), lambda i: (i, 0)),
        out_shape=jax.ShapeDtypeStruct((n, d), jnp.float32),
    )(flat, u, win3, w_top, w_bot, bias)


def kernel(expressions_encodings, symbols_encodings, expr_idx, token_idx, symbol_idx, W_z, b_z, W_h, b_h):
    b, l, d = expressions_encodings.shape
    n = b * l
    flat_idx = l * expr_idx.astype(jnp.int32) + token_idx.astype(jnp.int32)
    flat = expressions_encodings.reshape(n, d)
    neg1 = jnp.full((n,), -1, jnp.int32)

    tables = _winner_tables(flat_idx, symbol_idx.astype(jnp.int32), neg1, n)
    win_sym, idx = _combine_tables(tables, n)

    s_rows = symbols_encodings.shape[0]
    w = d // 2
    sym_b = symbols_encodings.astype(jnp.bfloat16)
    sym_i32 = jax.lax.bitcast_convert_type(sym_b.reshape(s_rows, w, 2), jnp.int32)
    u_i32 = _gather_symbols(sym_i32, idx, n, s_rows, w)
    u = jax.lax.bitcast_convert_type(u_i32, jnp.bfloat16).reshape(n, d)

    w_top = jnp.concatenate([W_z[:d], W_h[:d]], axis=1).astype(jnp.bfloat16)
    w_bot = jnp.concatenate([W_z[d:], W_h[d:]], axis=1).astype(jnp.bfloat16)
    bias = jnp.concatenate([b_z, b_h]).reshape(1, 2 * d)

    out = _fused_update(flat, u, win_sym, w_top, w_bot, bias)
    return out.reshape(b, l, d)


# speedup vs baseline: 1.0015x; 1.0015x over previous
import functools

import jax
import jax.numpy as jnp
from jax import lax
from jax.experimental import pallas as pl
from jax.experimental.pallas import tpu as pltpu
from jax.experimental.pallas import tpu_sc as plsc

_NC = 2
_NS = 16
_NW = _NC * _NS
_LANES = 16


def _mesh():
    return plsc.VectorSubcoreMesh(
        core_axis_name="c", subcore_axis_name="s", num_cores=_NC, num_subcores=_NS
    )


def _winner_tables(flat_idx, symbol_idx, neg1, n_rows):
    e = flat_idx.shape[0]
    chunk = e // _NW

    @functools.partial(
        pl.kernel,
        out_type=jax.ShapeDtypeStruct((_NW, n_rows), jnp.int32),
        mesh=_mesh(),
        compiler_params=pltpu.CompilerParams(needs_layout_passes=False),
        scratch_types=[
            pltpu.VMEM((chunk,), jnp.int32),
            pltpu.VMEM((chunk,), jnp.int32),
            pltpu.VMEM((n_rows,), jnp.int32),
            pltpu.VMEM((2 * _LANES,), jnp.int32),
        ],
    )
    def k(fi_hbm, si_hbm, neg1_hbm, tables_hbm, fi_v, si_v, tbl_v, shf_v):
        wid = lax.axis_index("s") * _NC + lax.axis_index("c")
        base = wid * chunk
        pltpu.sync_copy(fi_hbm.at[pl.ds(base, chunk)], fi_v)
        pltpu.sync_copy(si_hbm.at[pl.ds(base, chunk)], si_v)
        pltpu.sync_copy(neg1_hbm, tbl_v)
        lanes = lax.iota(jnp.int32, _LANES)
        shf_v[pl.ds(_LANES, _LANES)] = jnp.zeros((_LANES,), jnp.int32)

        def body(g, carry):
            off = g * _LANES
            fi = fi_v[pl.ds(off, _LANES)]
            si = si_v[pl.ds(off, _LANES)]
            key = fi * chunk + (off + lanes)
            skey, sval = plsc.sort_key_val(key, si)
            row = skey // chunk
            shf_v[pl.ds(0, _LANES)] = row
            nxt = shf_v[pl.ds(1, _LANES)]
            surv = (row != nxt) | (lanes == _LANES - 1)
            plsc.store_scatter(tbl_v, [row], sval, mask=surv)
            return carry

        lax.fori_loop(0, chunk // _LANES, body, 0)
        pltpu.sync_copy(tbl_v, tables_hbm.at[wid])

    return k(flat_idx, symbol_idx, neg1)


def _combine_tables(tables, n_rows):
    rows_per_w = n_rows // _NW

    @functools.partial(
        pl.kernel,
        out_type=(
            jax.ShapeDtypeStruct((n_rows,), jnp.int32),
            jax.ShapeDtypeStruct((n_rows,), jnp.int32),
        ),
        mesh=_mesh(),
        scratch_types=[
            pltpu.VMEM((_NW, rows_per_w), jnp.int32),
            pltpu.VMEM((rows_per_w,), jnp.int32),
            pltpu.VMEM((rows_per_w,), jnp.int32),
        ],
    )
    def k(tables_hbm, win_hbm, idx_hbm, tbuf, win_v, idx_v):
        wid = lax.axis_index("s") * _NC + lax.axis_index("c")
        rbase = wid * rows_per_w
        pltpu.sync_copy(tables_hbm.at[:, pl.ds(rbase, rows_per_w)], tbuf)

        def body(g, carry):
            acc = jnp.full((_LANES,), -1, jnp.int32)
            for c in range(_NW):
                t = tbuf[c, pl.ds(g * _LANES, _LANES)]
                acc = jnp.where(t >= 0, t, acc)
            win_v[pl.ds(g * _LANES, _LANES)] = acc
            idx_v[pl.ds(g * _LANES, _LANES)] = jnp.maximum(acc, 0)
            return carry

        lax.fori_loop(0, rows_per_w // _LANES, body, 0)
        pltpu.sync_copy(win_v, win_hbm.at[pl.ds(rbase, rows_per_w)])
        pltpu.sync_copy(idx_v, idx_hbm.at[pl.ds(rbase, rows_per_w)])

    return k(tables)


def _gather_symbols(symi, idx, n_rows, s_rows, w):
    rows_per_w = n_rows // _NW
    gb = 128
    ring = 2
    nb = rows_per_w // gb

    @functools.partial(
        pl.kernel,
        out_type=jax.ShapeDtypeStruct((n_rows, w), jnp.int32),
        mesh=_mesh(),
        scratch_types=[
            pltpu.VMEM((rows_per_w,), jnp.int32),
            [pltpu.VMEM((gb, w), jnp.int32) for _ in range(ring)],
            [pltpu.SemaphoreType.DMA for _ in range(ring)],
            pltpu.SemaphoreType.DMA,
        ],
    )
    def k(symi_hbm, idx_hbm, u_hbm, idx_v, bufs, gsems, osem):
        wid = lax.axis_index("s") * _NC + lax.axis_index("c")
        rbase = wid * rows_per_w
        pltpu.sync_copy(idx_hbm.at[pl.ds(rbase, rows_per_w)], idx_v)

        def gstart(g, j):
            pltpu.async_copy(
                symi_hbm.at[idx_v.at[pl.ds(g * gb, gb)]], bufs[j], gsems[j]
            )

        def gwait(j):
            pltpu.make_async_copy(
                symi_hbm.at[idx_v.at[pl.ds(0, gb)]], bufs[j], gsems[j]
            ).wait()

        def ostart(g, j):
            pltpu.async_copy(bufs[j], u_hbm.at[pl.ds(rbase + g * gb, gb)], osem)

        def owait(j):
            pltpu.make_async_copy(bufs[j], u_hbm.at[pl.ds(rbase, gb)], osem).wait()

        for j in range(ring - 1):
            gstart(j, j)

        def pbody(i, carry):
            for j in range(ring):
                g = ring * i + j
                jp = (j + ring - 1) % ring
                if j == 0:
                    @pl.when(i > 0)
                    def _():
                        owait(jp)
                else:
                    owait(jp)

                @pl.when(g + ring - 1 < nb)
                def _():
                    gstart(g + ring - 1, jp)

                gwait(j)
                ostart(g, j)
            return carry

        lax.fori_loop(0, nb // ring, pbody, 0)
        owait(ring - 1)

    return k(symi, idx)


def _fused_body(flat_ref, u_ref, win_ref, wtop_ref, wbot_ref, bias_ref, out_ref):
    prev = flat_ref[...]
    zh = (
        jnp.dot(prev.astype(jnp.bfloat16), wtop_ref[...], preferred_element_type=jnp.float32)
        + jnp.dot(u_ref[...], wbot_ref[...], preferred_element_type=jnp.float32)
        + bias_ref[...]
    )
    d = prev.shape[-1]
    z = jax.nn.sigmoid(zh[:, :d])
    h = jnp.tanh(zh[:, d:])
    blk = prev.shape[0]
    maskf = (win_ref[0, 0, :] >= 0).astype(jnp.float32).reshape(blk, 1)
    out_ref[...] = prev + (maskf * z) * (h - prev)


def _fused_update(flat, u, win_sym, w_top, w_bot, bias, blk=512):
    n, d = flat.shape
    grid = n // blk
    win3 = win_sym.reshape(grid, 1, blk)
    return pl.pallas_call(
        _fused_body,
        grid=(grid,),
        in_specs=[
            pl.BlockSpec((blk, d), lambda i: (i, 0)),
            pl.BlockSpec((blk, d), lambda i: (i, 0)),
            pl.BlockSpec((1, 1, blk), lambda i: (i, 0, 0)),
            pl.BlockSpec((d, 2 * d), lambda i: (0, 0)),
            pl.BlockSpec((d, 2 * d), lambda i: (0, 0)),
            pl.BlockSpec((1, 2 * d), lambda i: (0, 0)),
        ],
        out_specs=pl.BlockSpec((blk, d), lambda i: (i, 0)),
        out_shape=jax.ShapeDtypeStruct((n, d), jnp.float32),
    )(flat, u, win3, w_top, w_bot, bias)


def kernel(expressions_encodings, symbols_encodings, expr_idx, token_idx, symbol_idx, W_z, b_z, W_h, b_h):
    b, l, d = expressions_encodings.shape
    n = b * l
    flat_idx = l * expr_idx.astype(jnp.int32) + token_idx.astype(jnp.int32)
    flat = expressions_encodings.reshape(n, d)
    neg1 = jnp.full((n,), -1, jnp.int32)

    tables = _winner_tables(flat_idx, symbol_idx.astype(jnp.int32), neg1, n)
    win_sym, idx = _combine_tables(tables, n)

    s_rows = symbols_encodings.shape[0]
    w = d // 2
    sym_b = symbols_encodings.astype(jnp.bfloat16)
    sym_i32 = jax.lax.bitcast_convert_type(sym_b.reshape(s_rows, w, 2), jnp.int32)
    u_i32 = _gather_symbols(sym_i32, idx, n, s_rows, w)
    u = jax.lax.bitcast_convert_type(u_i32, jnp.bfloat16).reshape(n, d)

    w_top = jnp.concatenate([W_z[:d], W_h[:d]], axis=1).astype(jnp.bfloat16)
    w_bot = jnp.concatenate([W_z[d:], W_h[d:]], axis=1).astype(jnp.bfloat16)
    bias = jnp.concatenate([b_z, b_h]).reshape(1, 2 * d)

    out = _fused_update(flat, u, win_sym, w_top, w_bot, bias)
    return out.reshape(b, l, d)


# speedup vs baseline: 1.4611x; 1.4589x over previous
import functools

import jax
import jax.numpy as jnp
from jax import lax
from jax.experimental import pallas as pl
from jax.experimental.pallas import tpu as pltpu
from jax.experimental.pallas import tpu_sc as plsc

_NC = 2
_NS = 16
_NW = _NC * _NS
_LANES = 16


def _mesh():
    return plsc.VectorSubcoreMesh(
        core_axis_name="c", subcore_axis_name="s", num_cores=_NC, num_subcores=_NS
    )


def _winner_tables(flat_idx, symbol_idx, neg1, n_rows):
    e = flat_idx.shape[0]
    chunk = e // _NW

    @functools.partial(
        pl.kernel,
        out_type=jax.ShapeDtypeStruct((_NW, n_rows), jnp.int32),
        mesh=_mesh(),
        compiler_params=pltpu.CompilerParams(needs_layout_passes=False),
        scratch_types=[
            pltpu.VMEM((chunk,), jnp.int32),
            pltpu.VMEM((chunk,), jnp.int32),
            pltpu.VMEM((n_rows,), jnp.int32),
            pltpu.VMEM((2 * _LANES,), jnp.int32),
        ],
    )
    def k(fi_hbm, si_hbm, neg1_hbm, tables_hbm, fi_v, si_v, tbl_v, shf_v):
        wid = lax.axis_index("s") * _NC + lax.axis_index("c")
        base = wid * chunk
        pltpu.sync_copy(fi_hbm.at[pl.ds(base, chunk)], fi_v)
        pltpu.sync_copy(si_hbm.at[pl.ds(base, chunk)], si_v)
        pltpu.sync_copy(neg1_hbm, tbl_v)
        lanes = lax.iota(jnp.int32, _LANES)
        shf_v[pl.ds(_LANES, _LANES)] = jnp.zeros((_LANES,), jnp.int32)

        def body(g, carry):
            off = g * _LANES
            fi = fi_v[pl.ds(off, _LANES)]
            si = si_v[pl.ds(off, _LANES)]
            key = fi * chunk + (off + lanes)
            skey, sval = plsc.sort_key_val(key, si)
            row = skey // chunk
            shf_v[pl.ds(0, _LANES)] = row
            nxt = shf_v[pl.ds(1, _LANES)]
            surv = (row != nxt) | (lanes == _LANES - 1)
            plsc.store_scatter(tbl_v, [row], sval, mask=surv)
            return carry

        lax.fori_loop(0, chunk // _LANES, body, 0)
        pltpu.sync_copy(tbl_v, tables_hbm.at[wid])

    return k(flat_idx, symbol_idx, neg1)


def _combine_tables(tables, n_rows):
    rows_per_w = n_rows // _NW

    @functools.partial(
        pl.kernel,
        out_type=(
            jax.ShapeDtypeStruct((n_rows,), jnp.int32),
            jax.ShapeDtypeStruct((n_rows,), jnp.int32),
        ),
        mesh=_mesh(),
        scratch_types=[
            pltpu.VMEM((_NW, rows_per_w), jnp.int32),
            pltpu.VMEM((rows_per_w,), jnp.int32),
            pltpu.VMEM((rows_per_w,), jnp.int32),
        ],
    )
    def k(tables_hbm, win_hbm, idx_hbm, tbuf, win_v, idx_v):
        wid = lax.axis_index("s") * _NC + lax.axis_index("c")
        rbase = wid * rows_per_w
        pltpu.sync_copy(tables_hbm.at[:, pl.ds(rbase, rows_per_w)], tbuf)

        def body(g, carry):
            acc = jnp.full((_LANES,), -1, jnp.int32)
            for c in range(_NW):
                t = tbuf[c, pl.ds(g * _LANES, _LANES)]
                acc = jnp.where(t >= 0, t, acc)
            win_v[pl.ds(g * _LANES, _LANES)] = acc
            idx_v[pl.ds(g * _LANES, _LANES)] = jnp.maximum(acc, 0)
            return carry

        lax.fori_loop(0, rows_per_w // _LANES, body, 0)
        pltpu.sync_copy(win_v, win_hbm.at[pl.ds(rbase, rows_per_w)])
        pltpu.sync_copy(idx_v, idx_hbm.at[pl.ds(rbase, rows_per_w)])

    return k(tables)


def _gather_symbols(symi, idx, n_rows, s_rows, w):
    rows_per_w = n_rows // _NW
    gb = 64
    ring = 2
    nb = rows_per_w // gb

    @functools.partial(
        pl.kernel,
        out_type=jax.ShapeDtypeStruct((n_rows, w), jnp.int32),
        mesh=_mesh(),
        scratch_types=[
            pltpu.VMEM((rows_per_w,), jnp.int32),
            [pltpu.VMEM((gb, w), jnp.int32) for _ in range(ring)],
            [pltpu.SemaphoreType.DMA for _ in range(ring)],
            pltpu.SemaphoreType.DMA,
        ],
    )
    def k(symi_hbm, idx_hbm, u_hbm, idx_v, bufs, gsems, osem):
        wid = lax.axis_index("s") * _NC + lax.axis_index("c")
        rbase = wid * rows_per_w
        pltpu.sync_copy(idx_hbm.at[pl.ds(rbase, rows_per_w)], idx_v)

        def gstart(g, j):
            pltpu.async_copy(
                symi_hbm.at[idx_v.at[pl.ds(g * gb, gb)]], bufs[j], gsems[j]
            )

        def gwait(j):
            pltpu.make_async_copy(
                symi_hbm.at[idx_v.at[pl.ds(0, gb)]], bufs[j], gsems[j]
            ).wait()

        def ostart(g, j):
            pltpu.async_copy(bufs[j], u_hbm.at[pl.ds(rbase + g * gb, gb)], osem)

        def owait(j):
            pltpu.make_async_copy(bufs[j], u_hbm.at[pl.ds(rbase, gb)], osem).wait()

        for j in range(ring - 1):
            gstart(j, j)

        def pbody(i, carry):
            for j in range(ring):
                g = ring * i + j
                jp = (j + ring - 1) % ring
                if j == 0:
                    @pl.when(i > 0)
                    def _():
                        owait(jp)
                else:
                    owait(jp)

                @pl.when(g + ring - 1 < nb)
                def _():
                    gstart(g + ring - 1, jp)

                gwait(j)
                ostart(g, j)
            return carry

        lax.fori_loop(0, nb // ring, pbody, 0)
        owait(ring - 1)

    return k(symi, idx)


def _fused_body(flat_ref, u_ref, win_ref, wtop_ref, wbot_ref, bias_ref, out_ref):
    prev = flat_ref[...]
    zh = (
        jnp.dot(prev.astype(jnp.bfloat16), wtop_ref[...], preferred_element_type=jnp.float32)
        + jnp.dot(u_ref[...].astype(jnp.bfloat16), wbot_ref[...], preferred_element_type=jnp.float32)
        + bias_ref[...]
    )
    d = prev.shape[-1]
    z = jax.nn.sigmoid(zh[:, :d])
    h = jnp.tanh(zh[:, d:])
    blk = prev.shape[0]
    maskf = (win_ref[0, 0, :] >= 0).astype(jnp.float32).reshape(blk, 1)
    out_ref[...] = prev + (maskf * z) * (h - prev)


def _fused_update(flat, u, win_sym, w_top, w_bot, bias, blk=512):
    n, d = flat.shape
    grid = n // blk
    win3 = win_sym.reshape(grid, 1, blk)
    return pl.pallas_call(
        _fused_body,
        grid=(grid,),
        in_specs=[
            pl.BlockSpec((blk, d), lambda i: (i, 0)),
            pl.BlockSpec((blk, d), lambda i: (i, 0)),
            pl.BlockSpec((1, 1, blk), lambda i: (i, 0, 0)),
            pl.BlockSpec((d, 2 * d), lambda i: (0, 0)),
            pl.BlockSpec((d, 2 * d), lambda i: (0, 0)),
            pl.BlockSpec((1, 2 * d), lambda i: (0, 0)),
        ],
        out_specs=pl.BlockSpec((blk, d), lambda i: (i, 0)),
        out_shape=jax.ShapeDtypeStruct((n, d), jnp.float32),
    )(flat, u, win3, w_top, w_bot, bias)


def kernel(expressions_encodings, symbols_encodings, expr_idx, token_idx, symbol_idx, W_z, b_z, W_h, b_h):
    b, l, d = expressions_encodings.shape
    n = b * l
    flat_idx = l * expr_idx.astype(jnp.int32) + token_idx.astype(jnp.int32)
    flat = expressions_encodings.reshape(n, d)
    neg1 = jnp.full((n,), -1, jnp.int32)

    tables = _winner_tables(flat_idx, symbol_idx.astype(jnp.int32), neg1, n)
    win_sym, idx = _combine_tables(tables, n)

    s_rows = symbols_encodings.shape[0]
    w = d
    sym_i32 = jax.lax.bitcast_convert_type(symbols_encodings, jnp.int32)
    u_i32 = _gather_symbols(sym_i32, idx, n, s_rows, w)
    u = jax.lax.bitcast_convert_type(u_i32, jnp.float32)

    w_top = jnp.concatenate([W_z[:d], W_h[:d]], axis=1).astype(jnp.bfloat16)
    w_bot = jnp.concatenate([W_z[d:], W_h[d:]], axis=1).astype(jnp.bfloat16)
    bias = jnp.concatenate([b_z, b_h]).reshape(1, 2 * d)

    out = _fused_update(flat, u, win_sym, w_top, w_bot, bias)
    return out.reshape(b, l, d)


# speedup vs baseline: 1.7151x; 1.1738x over previous
import functools

import jax
import jax.numpy as jnp
from jax import lax
from jax.experimental import pallas as pl
from jax.experimental.pallas import tpu as pltpu
from jax.experimental.pallas import tpu_sc as plsc

_NC = 2
_NS = 16
_NW = _NC * _NS
_LANES = 16


def _mesh():
    return plsc.VectorSubcoreMesh(
        core_axis_name="c", subcore_axis_name="s", num_cores=_NC, num_subcores=_NS
    )


def _winner_tables(flat_idx, symbol_idx, neg1, n_rows):
    e = flat_idx.shape[0]
    chunk = e // _NW

    @functools.partial(
        pl.kernel,
        out_type=jax.ShapeDtypeStruct((_NW, n_rows), jnp.int32),
        mesh=_mesh(),
        compiler_params=pltpu.CompilerParams(needs_layout_passes=False),
        scratch_types=[
            pltpu.VMEM((chunk,), jnp.int32),
            pltpu.VMEM((chunk,), jnp.int32),
            pltpu.VMEM((n_rows,), jnp.int32),
            pltpu.VMEM((2 * _LANES,), jnp.int32),
        ],
    )
    def k(fi_hbm, si_hbm, neg1_hbm, tables_hbm, fi_v, si_v, tbl_v, shf_v):
        wid = lax.axis_index("s") * _NC + lax.axis_index("c")
        base = wid * chunk
        pltpu.sync_copy(fi_hbm.at[pl.ds(base, chunk)], fi_v)
        pltpu.sync_copy(si_hbm.at[pl.ds(base, chunk)], si_v)
        pltpu.sync_copy(neg1_hbm, tbl_v)
        lanes = lax.iota(jnp.int32, _LANES)
        shf_v[pl.ds(_LANES, _LANES)] = jnp.zeros((_LANES,), jnp.int32)

        def body(g, carry):
            off = g * _LANES
            fi = fi_v[pl.ds(off, _LANES)]
            si = si_v[pl.ds(off, _LANES)]
            key = fi * chunk + (off + lanes)
            skey, sval = plsc.sort_key_val(key, si)
            row = skey // chunk
            shf_v[pl.ds(0, _LANES)] = row
            nxt = shf_v[pl.ds(1, _LANES)]
            surv = (row != nxt) | (lanes == _LANES - 1)
            plsc.store_scatter(tbl_v, [row], sval, mask=surv)
            return carry

        lax.fori_loop(0, chunk // _LANES, body, 0)
        pltpu.sync_copy(tbl_v, tables_hbm.at[wid])

    return k(flat_idx, symbol_idx, neg1)


def _combine_tables(tables, n_rows):
    rows_per_w = n_rows // _NW

    @functools.partial(
        pl.kernel,
        out_type=(
            jax.ShapeDtypeStruct((n_rows,), jnp.int32),
            jax.ShapeDtypeStruct((n_rows,), jnp.int32),
        ),
        mesh=_mesh(),
        scratch_types=[
            pltpu.VMEM((_NW, rows_per_w), jnp.int32),
            pltpu.VMEM((rows_per_w,), jnp.int32),
            pltpu.VMEM((rows_per_w,), jnp.int32),
        ],
    )
    def k(tables_hbm, win_hbm, idx_hbm, tbuf, win_v, idx_v):
        wid = lax.axis_index("s") * _NC + lax.axis_index("c")
        rbase = wid * rows_per_w
        pltpu.sync_copy(tables_hbm.at[:, pl.ds(rbase, rows_per_w)], tbuf)

        def body(g, carry):
            acc = jnp.full((_LANES,), -1, jnp.int32)
            for c in range(_NW):
                t = tbuf[c, pl.ds(g * _LANES, _LANES)]
                acc = jnp.where(t >= 0, t, acc)
            win_v[pl.ds(g * _LANES, _LANES)] = acc
            idx_v[pl.ds(g * _LANES, _LANES)] = jnp.maximum(acc, 0)
            return carry

        lax.fori_loop(0, rows_per_w // _LANES, body, 0)
        pltpu.sync_copy(win_v, win_hbm.at[pl.ds(rbase, rows_per_w)])
        pltpu.sync_copy(idx_v, idx_hbm.at[pl.ds(rbase, rows_per_w)])

    return k(tables)


def _gather_symbols(symi, idx, n_rows, s_rows, w):
    rows_per_w = n_rows // _NW
    gb = 128
    ring = 2
    nb = rows_per_w // gb

    @functools.partial(
        pl.kernel,
        out_type=jax.ShapeDtypeStruct((n_rows, w), jnp.int32),
        mesh=_mesh(),
        scratch_types=[
            pltpu.VMEM((rows_per_w,), jnp.int32),
            [pltpu.VMEM((gb, w), jnp.int32) for _ in range(ring)],
            [pltpu.SemaphoreType.DMA for _ in range(ring)],
            pltpu.SemaphoreType.DMA,
        ],
    )
    def k(symi_hbm, idx_hbm, u_hbm, idx_v, bufs, gsems, osem):
        wid = lax.axis_index("s") * _NC + lax.axis_index("c")
        rbase = wid * rows_per_w
        pltpu.sync_copy(idx_hbm.at[pl.ds(rbase, rows_per_w)], idx_v)

        def gstart(g, j):
            pltpu.async_copy(
                symi_hbm.at[idx_v.at[pl.ds(g * gb, gb)]], bufs[j], gsems[j]
            )

        def gwait(j):
            pltpu.make_async_copy(
                symi_hbm.at[idx_v.at[pl.ds(0, gb)]], bufs[j], gsems[j]
            ).wait()

        def ostart(g, j):
            pltpu.async_copy(bufs[j], u_hbm.at[pl.ds(rbase + g * gb, gb)], osem)

        def owait(j):
            pltpu.make_async_copy(bufs[j], u_hbm.at[pl.ds(rbase, gb)], osem).wait()

        for j in range(ring - 1):
            gstart(j, j)

        def pbody(i, carry):
            for j in range(ring):
                g = ring * i + j
                jp = (j + ring - 1) % ring
                if j == 0:
                    @pl.when(i > 0)
                    def _():
                        owait(jp)
                else:
                    owait(jp)

                @pl.when(g + ring - 1 < nb)
                def _():
                    gstart(g + ring - 1, jp)

                gwait(j)
                ostart(g, j)
            return carry

        lax.fori_loop(0, nb // ring, pbody, 0)
        owait(ring - 1)

    return k(symi, idx)


def _fused_body(flat_ref, u_ref, win_ref, wtop_ref, wbote_ref, wboto_ref, bias_ref, out_ref):
    prev = flat_ref[...]
    w32 = u_ref[...]
    ueven = pltpu.bitcast(w32 << 16, jnp.float32).astype(jnp.bfloat16)
    uodd = pltpu.bitcast(w32 & jnp.int32(-65536), jnp.float32).astype(jnp.bfloat16)
    zh = (
        jnp.dot(prev.astype(jnp.bfloat16), wtop_ref[...], preferred_element_type=jnp.float32)
        + jnp.dot(ueven, wbote_ref[...], preferred_element_type=jnp.float32)
        + jnp.dot(uodd, wboto_ref[...], preferred_element_type=jnp.float32)
        + bias_ref[...]
    )
    d = prev.shape[-1]
    z = jax.nn.sigmoid(zh[:, :d])
    h = jnp.tanh(zh[:, d:])
    blk = prev.shape[0]
    maskf = (win_ref[0, 0, :] >= 0).astype(jnp.float32).reshape(blk, 1)
    out_ref[...] = prev + (maskf * z) * (h - prev)


def _fused_update(flat, u_i32, win_sym, w_top, w_bote, w_boto, bias, blk=512):
    n, d = flat.shape
    h = d // 2
    grid = n // blk
    win3 = win_sym.reshape(grid, 1, blk)
    return pl.pallas_call(
        _fused_body,
        grid=(grid,),
        in_specs=[
            pl.BlockSpec((blk, d), lambda i: (i, 0)),
            pl.BlockSpec((blk, h), lambda i: (i, 0)),
            pl.BlockSpec((1, 1, blk), lambda i: (i, 0, 0)),
            pl.BlockSpec((d, 2 * d), lambda i: (0, 0)),
            pl.BlockSpec((h, 2 * d), lambda i: (0, 0)),
            pl.BlockSpec((h, 2 * d), lambda i: (0, 0)),
            pl.BlockSpec((1, 2 * d), lambda i: (0, 0)),
        ],
        out_specs=pl.BlockSpec((blk, d), lambda i: (i, 0)),
        out_shape=jax.ShapeDtypeStruct((n, d), jnp.float32),
    )(flat, u_i32, win3, w_top, w_bote, w_boto, bias)


def kernel(expressions_encodings, symbols_encodings, expr_idx, token_idx, symbol_idx, W_z, b_z, W_h, b_h):
    b, l, d = expressions_encodings.shape
    n = b * l
    flat_idx = l * expr_idx.astype(jnp.int32) + token_idx.astype(jnp.int32)
    flat = expressions_encodings.reshape(n, d)
    neg1 = jnp.full((n,), -1, jnp.int32)

    tables = _winner_tables(flat_idx, symbol_idx.astype(jnp.int32), neg1, n)
    win_sym, idx = _combine_tables(tables, n)

    s_rows = symbols_encodings.shape[0]
    w = d // 2
    sym_b = symbols_encodings.astype(jnp.bfloat16)
    sym_i32 = jax.lax.bitcast_convert_type(sym_b.reshape(s_rows, w, 2), jnp.int32)
    u_i32 = _gather_symbols(sym_i32, idx, n, s_rows, w)

    w_top = jnp.concatenate([W_z[:d], W_h[:d]], axis=1).astype(jnp.bfloat16)
    w_bot = jnp.concatenate([W_z[d:], W_h[d:]], axis=1).astype(jnp.bfloat16)
    bias = jnp.concatenate([b_z, b_h]).reshape(1, 2 * d)

    out = _fused_update(flat, u_i32, win_sym, w_top, w_bot[0::2], w_bot[1::2], bias)
    return out.reshape(b, l, d)
